# baseline (speedup 1.0000x reference)
import functools

import jax
import jax.numpy as jnp
from jax import lax
from jax.experimental import pallas as pl
from jax.experimental.pallas import tpu as pltpu
from jax.experimental.pallas import tpu_sc as plsc

EMBED_DIM = 16
HIST_LEN = 200
CHUNK = 16
NUM_ACTIVE = -(-HIST_LEN // CHUNK)
OVERLAP = NUM_ACTIVE * CHUNK - HIST_LEN

_mesh = plsc.VectorSubcoreMesh(
    core_axis_name="c", subcore_axis_name="s", num_cores=1)


@functools.partial(
    pl.kernel,
    mesh=_mesh,
    compiler_params=pltpu.CompilerParams(
        needs_layout_passes=False, skip_device_barrier=True),
    out_type=[
        jax.ShapeDtypeStruct((2, EMBED_DIM), jnp.float32),
        jax.ShapeDtypeStruct((NUM_ACTIVE, EMBED_DIM), jnp.float32),
    ],
    scratch_types=[
        pltpu.VMEM((CHUNK, EMBED_DIM, 128), jnp.float32),
        pltpu.VMEM((CHUNK,), jnp.int32),
        pltpu.VMEM((EMBED_DIM,), jnp.float32),
        pltpu.VMEM((NUM_ACTIVE, EMBED_DIM), jnp.float32),
        pltpu.VMEM((EMBED_DIM,), jnp.float32),
        pltpu.VMEM((EMBED_DIM,), jnp.float32),
        pltpu.VMEM((2, EMBED_DIM), jnp.float32),
        pltpu.SemaphoreType.DMA,
        pltpu.SemaphoreType.DMA,
        pltpu.SemaphoreType.DMA,
    ],
)
def _ehr_lr_sc(idx_hbm, tableT_hbm, w_hbm, b_hbm, out_hbm,
               parts_hbm, blocks_v, myidx_v, part_v, parts_v, wv, bv, outv,
               sem, sem_w, sem_b):
    sid = lax.axis_index("s")

    @pl.when(sid == 0)
    def _():
        pltpu.async_copy(w_hbm, wv, sem_w)
        pltpu.async_copy(b_hbm, bv.at[pl.ds(0, 1)], sem_b)

    @pl.when(sid < NUM_ACTIVE)
    def _():
        for ci in range(NUM_ACTIVE):
            base_ci = min(ci * CHUNK, HIST_LEN - CHUNK)

            @pl.when(sid == ci)
            def _(base_ci=base_ci):
                pltpu.sync_copy(idx_hbm.at[pl.ds(base_ci, CHUNK)], myidx_v)

        v16 = myidx_v[...]
        lanes = lax.iota(jnp.int32, EMBED_DIM)
        accs = [jnp.zeros((EMBED_DIM,), jnp.float32) for _ in range(4)]
        last = sid == NUM_ACTIVE - 1
        copies = []
        for j in range(CHUNK):
            r = v16[j]
            blk = pl.multiple_of((r // 128) * 128, 128)
            copies.append(pltpu.async_copy(
                tableT_hbm.at[:, pl.ds(blk, 128)], blocks_v.at[j], sem))
        for c in copies:
            c.wait()
        for j in range(CHUNK):
            r = v16[j]
            lane = jnp.broadcast_to(lax.rem(r, 128), (EMBED_DIM,))
            col = plsc.load_gather(blocks_v.at[j], [lanes, lane])
            valid = jnp.logical_or(jnp.logical_not(last),
                                   j >= OVERLAP).astype(jnp.float32)
            accs[j % 4] = accs[j % 4] + col * valid
        part_v[...] = (accs[0] + accs[1]) + (accs[2] + accs[3])
        pltpu.sync_copy(part_v, parts_hbm.at[sid])

    plsc.subcore_barrier()

    @pl.when(sid == 0)
    def _():
        pltpu.sync_copy(parts_hbm, parts_v)
        acc = parts_v[0]
        for t in range(1, NUM_ACTIVE):
            acc = acc + parts_v[t]
        outv[1, :] = acc

        pltpu.make_async_copy(w_hbm, wv, sem_w).wait()
        pltpu.make_async_copy(b_hbm, bv.at[pl.ds(0, 1)], sem_b).wait()
        t = acc * wv[...]
        lanes = lax.iota(jnp.int32, EMBED_DIM)
        dnums = lax.GatherDimensionNumbers(
            offset_dims=(), collapsed_slice_dims=(0,), start_index_map=(0,))
        for k in (1, 2, 4, 8):
            shuf = lax.gather(
                t, (lanes ^ k)[:, None], dnums, (1,),
                mode=lax.GatherScatterMode.PROMISE_IN_BOUNDS)
            t = t + shuf
        z = t + bv[...]
        outv[0, :] = 1.0 / (1.0 + jnp.exp(-z))

        pltpu.sync_copy(outv, out_hbm)


def kernel(label, ehr_seq, emb, W, b):
    idx = ehr_seq.astype(jnp.int32)
    out2, _ = _ehr_lr_sc(idx, emb.T, W.reshape(EMBED_DIM), b)
    output = out2[0, :1].reshape(1, 1)
    embedded = out2[1:2, :].reshape(1, EMBED_DIM)
    return (output, label, embedded)


# speedup vs baseline: 1.0765x; 1.0765x over previous
import functools

import jax
import jax.numpy as jnp
from jax import lax
from jax.experimental import pallas as pl
from jax.experimental.pallas import tpu as pltpu
from jax.experimental.pallas import tpu_sc as plsc

EMBED_DIM = 16
HIST_LEN = 200
CHUNK = 16
NUM_ACTIVE = -(-HIST_LEN // CHUNK)
OVERLAP = NUM_ACTIVE * CHUNK - HIST_LEN

_mesh = plsc.VectorSubcoreMesh(
    core_axis_name="c", subcore_axis_name="s", num_cores=1)


@functools.partial(
    pl.kernel,
    mesh=_mesh,
    compiler_params=pltpu.CompilerParams(
        needs_layout_passes=False, skip_device_barrier=True),
    out_type=[
        jax.ShapeDtypeStruct((EMBED_DIM,), jnp.float32),
        jax.ShapeDtypeStruct((EMBED_DIM,), jnp.float32),
        jax.ShapeDtypeStruct((NUM_ACTIVE, EMBED_DIM), jnp.float32),
    ],
    scratch_types=[
        pltpu.VMEM((CHUNK, EMBED_DIM, 128), jnp.float32),
        pltpu.VMEM((CHUNK,), jnp.int32),
        pltpu.VMEM((EMBED_DIM,), jnp.float32),
        pltpu.VMEM((NUM_ACTIVE, EMBED_DIM), jnp.float32),
        pltpu.VMEM((EMBED_DIM,), jnp.float32),
        pltpu.VMEM((EMBED_DIM,), jnp.float32),
        pltpu.VMEM((EMBED_DIM,), jnp.float32),
        pltpu.SemaphoreType.DMA,
        pltpu.SemaphoreType.DMA,
        pltpu.SemaphoreType.DMA,
    ],
)
def _ehr_lr_sc(idx_hbm, tableT_hbm, w_hbm, b_hbm, sig_hbm, emb_hbm,
               parts_hbm, blocks_v, myidx_v, part_v, parts_v, wv, bv, sigv,
               sem, sem_w, sem_b):
    sid = lax.axis_index("s")

    @pl.when(sid == 0)
    def _():
        pltpu.async_copy(w_hbm, wv, sem_w)
        pltpu.async_copy(b_hbm, bv.at[pl.ds(0, 1)], sem_b)

    @pl.when(sid < NUM_ACTIVE)
    def _():
        for ci in range(NUM_ACTIVE):
            base_ci = min(ci * CHUNK, HIST_LEN - CHUNK)

            @pl.when(sid == ci)
            def _(base_ci=base_ci):
                pltpu.sync_copy(idx_hbm.at[pl.ds(base_ci, CHUNK)], myidx_v)

        v16 = myidx_v[...]
        lanes = lax.iota(jnp.int32, EMBED_DIM)
        accs = [jnp.zeros((EMBED_DIM,), jnp.float32) for _ in range(4)]
        last = sid == NUM_ACTIVE - 1
        copies = []
        for j in range(CHUNK):
            r = v16[j]
            blk = pl.multiple_of((r // 128) * 128, 128)
            copies.append(pltpu.async_copy(
                tableT_hbm.at[:, pl.ds(blk, 128)], blocks_v.at[j], sem))
        for c in copies:
            c.wait()
        for j in range(CHUNK):
            r = v16[j]
            lane = jnp.broadcast_to(lax.rem(r, 128), (EMBED_DIM,))
            col = plsc.load_gather(blocks_v.at[j], [lanes, lane])
            valid = jnp.logical_or(jnp.logical_not(last),
                                   j >= OVERLAP).astype(jnp.float32)
            accs[j % 4] = accs[j % 4] + col * valid
        part_v[...] = (accs[0] + accs[1]) + (accs[2] + accs[3])
        pltpu.sync_copy(part_v, parts_hbm.at[sid])

    plsc.subcore_barrier()

    @pl.when(sid == 0)
    def _():
        pltpu.sync_copy(parts_hbm, parts_v)
        acc = parts_v[0]
        for t in range(1, NUM_ACTIVE):
            acc = acc + parts_v[t]
        part_v[...] = acc

        pltpu.make_async_copy(w_hbm, wv, sem_w).wait()
        pltpu.make_async_copy(b_hbm, bv.at[pl.ds(0, 1)], sem_b).wait()
        t = acc * wv[...]
        lanes = lax.iota(jnp.int32, EMBED_DIM)
        dnums = lax.GatherDimensionNumbers(
            offset_dims=(), collapsed_slice_dims=(0,), start_index_map=(0,))
        for k in (1, 2, 4, 8):
            shuf = lax.gather(
                t, (lanes ^ k)[:, None], dnums, (1,),
                mode=lax.GatherScatterMode.PROMISE_IN_BOUNDS)
            t = t + shuf
        z = t + bv[...]
        sigv[...] = 1.0 / (1.0 + jnp.exp(-z))

        pltpu.sync_copy(sigv, sig_hbm)
        pltpu.sync_copy(part_v, emb_hbm)


def kernel(label, ehr_seq, emb, W, b):
    idx = ehr_seq.astype(jnp.int32)
    sig16, emb16, _ = _ehr_lr_sc(idx, emb.T, W.reshape(EMBED_DIM), b)
    output = sig16[:1].reshape(1, 1)
    embedded = emb16.reshape(1, EMBED_DIM)
    return (output, label, embedded)
